# baseline (speedup 1.0000x reference)
import functools

import jax
import jax.numpy as jnp
from jax import lax
from jax.experimental import pallas as pl
from jax.experimental.pallas import tpu as pltpu
from jax.experimental.pallas import tpu_sc as plsc

NUM_EMBEDDINGS = 100000
EMBED_DIM = 128
BATCH = 4096
HIST_LEN = 50

_HP = 56
_NC = 2
_NS = 16
_NW = _NC * _NS
_BPW = BATCH // _NW
_NB2 = 8
_K = _BPW // _NB2


def _make_gather():
    mesh = plsc.VectorSubcoreMesh(core_axis_name="c", subcore_axis_name="s")

    @functools.partial(
        pl.kernel,
        mesh=mesh,
        out_type=jax.ShapeDtypeStruct((BATCH, HIST_LEN, EMBED_DIM),
                                      jnp.float32),
        scratch_types=[
            pltpu.VMEM((_BPW * _HP,), jnp.int32),
            pltpu.VMEM((2, _NB2, HIST_LEN, EMBED_DIM), jnp.float32),
            pltpu.SemaphoreType.DMA,
            pltpu.SemaphoreType.DMA,
        ],
    )
    def gather_kernel(idx_hbm, table_hbm, out_hbm, idx_v, rows_v, sem0, sem1):
        sems = [sem0, sem1]
        wid = lax.axis_index("s") * _NC + lax.axis_index("c")
        base_b = wid * _BPW
        pltpu.sync_copy(idx_hbm.at[pl.ds(base_b * _HP, _BPW * _HP)], idx_v)

        def g_start(q2, b):
            for h in range(_NB2):
                pltpu.async_copy(
                    table_hbm.at[idx_v.at[pl.ds((q2 * _NB2 + h) * _HP,
                                                HIST_LEN)]],
                    rows_v.at[b, h], sems[b])

        def g_wait(q2, b):
            pltpu.make_async_copy(
                out_hbm.at[pl.ds(base_b, _NB2)], rows_v.at[b],
                sems[b]).wait()

        def s_start(q2, b):
            pltpu.async_copy(rows_v.at[b],
                             out_hbm.at[pl.ds(base_b + q2 * _NB2, _NB2)],
                             sems[b])

        def s_wait(q2, b):
            pltpu.make_async_copy(rows_v.at[b],
                                  out_hbm.at[pl.ds(base_b + q2 * _NB2, _NB2)],
                                  sems[b]).wait()

        g_start(0, 0)
        g_wait(0, 0)
        s_start(0, 0)
        g_start(1, 1)

        def slot(t, b, b2):
            g_wait(t, b)
            s_start(t, b)
            s_wait(t - 1, b2)
            g_start(t + 1, b2)

        def body(t2, carry):
            slot(2 * t2 + 1, 1, 0)
            slot(2 * t2 + 2, 0, 1)
            return carry

        lax.fori_loop(0, (_K - 2) // 2, body, 0)

        b, b2 = (_K - 1) % 2, _K % 2
        g_wait(_K - 1, b)
        s_start(_K - 1, b)
        s_wait(_K - 2, b2)
        s_wait(_K - 1, b)

    return gather_kernel


_gather = _make_gather()


def kernel(input_, weight):
    idx = jnp.pad(input_.astype(jnp.int32), ((0, 0), (0, _HP - HIST_LEN)))
    return _gather(idx.reshape(-1), weight)


# speedup vs baseline: 1.8426x; 1.8426x over previous
import functools

import jax
import jax.numpy as jnp
from jax import lax
from jax.experimental import pallas as pl
from jax.experimental.pallas import tpu as pltpu
from jax.experimental.pallas import tpu_sc as plsc

NUM_EMBEDDINGS = 100000
EMBED_DIM = 128
BATCH = 4096
HIST_LEN = 50

_NC = 2
_NS = 16
_NW = _NC * _NS
_BPW = BATCH // _NW
_K = HIST_LEN

_NBUF = 5
_G = 3
_T = _K // _NBUF


def _make_gather():
    mesh = plsc.VectorSubcoreMesh(core_axis_name="c", subcore_axis_name="s")

    @functools.partial(
        pl.kernel,
        mesh=mesh,
        out_type=jax.ShapeDtypeStruct((HIST_LEN, BATCH, EMBED_DIM),
                                      jnp.float32),
        scratch_types=[
            pltpu.VMEM((_K, _BPW), jnp.int32),
            pltpu.VMEM((_NBUF, _BPW, EMBED_DIM), jnp.float32),
        ]
        + [pltpu.SemaphoreType.DMA] * _NBUF,
    )
    def gather_kernel(idx_hbm, table_hbm, out_hbm, idx_v, rows_v,
                      sem0, sem1, sem2, sem3, sem4):
        sems = [sem0, sem1, sem2, sem3, sem4]
        wid = lax.axis_index("s") * _NC + lax.axis_index("c")
        base_b = wid * _BPW
        pltpu.sync_copy(idx_hbm.at[:, pl.ds(base_b, _BPW)], idx_v)

        def g_start(j, b):
            pltpu.async_copy(table_hbm.at[idx_v.at[j]], rows_v.at[b], sems[b])

        def g_wait(j, b):
            pltpu.make_async_copy(
                table_hbm.at[idx_v.at[j]], rows_v.at[b], sems[b]).wait()

        def out_slice(j):
            return out_hbm.at[j, pl.ds(base_b, _BPW)]

        def s_start(j, b):
            pltpu.async_copy(rows_v.at[b], out_slice(j), sems[b])

        def s_wait(j, b):
            pltpu.make_async_copy(rows_v.at[b], out_slice(j), sems[b]).wait()

        for c in range(_G):
            g_start(c, c % _NBUF)

        def slot(j, b, c_static=None):
            g_wait(j, b)
            s_start(j, b)
            c = j + _G if c_static is None else c_static
            bc = (b + _G) % _NBUF
            if c_static is None or c_static >= _NBUF:
                s_wait(c - _NBUF, bc)
            g_start(c, bc)

        for b in range(_NBUF):
            slot(b, b, c_static=b + _G)

        def outer(t, carry):
            for b in range(_NBUF):
                slot(t * _NBUF + b, b)
            return carry

        lax.fori_loop(1, _T - 1, outer, 0)

        for b in range(_NBUF):
            j = (_T - 1) * _NBUF + b
            g_wait(j, b)
            s_start(j, b)
            c = j + _G
            if c < _K:
                s_wait(c - _NBUF, c % _NBUF)
                g_start(c, c % _NBUF)

        for j in range(_K - _NBUF, _K):
            s_wait(j, j % _NBUF)

    return gather_kernel


_gather = _make_gather()


def kernel(input_, weight):
    idx_t = input_.astype(jnp.int32).T
    out = _gather(idx_t, weight)
    return out.transpose(1, 0, 2)
